# Initial kernel scaffold; baseline (speedup 1.0000x reference)
import jax
import jax.numpy as jnp
from jax.experimental import pallas as pl


def kernel(users_feature, items_feature, bundles_feature, deg_proj_W, size_pe_table, ui_edges, ub_edges, bundle_items, bundle_size):
    raise NotImplementedError("write your pallas kernel here")


# speedup vs baseline: 9.0485x; 9.0485x over previous
import functools
import math

import jax
import jax.numpy as jnp
from jax import lax
from jax.experimental import pallas as pl
from jax.experimental.pallas import tpu as pltpu
from jax.experimental.pallas import tpu_sc as plsc

NU = 100000
NI = 50000
NB = 20000
D = 64
NT = 20
W_DEG = 0.05
W_SIZE = 0.05

NCORES = 2
NTILES = 16
_BR = 400
_WH = 2048
_WE = 256
_WB = 32

_OFF_U_UI = 0
_OFF_I_UI = NU
_OFF_I_BI = NU + NI
_OFF_U_UB = NU + 2 * NI
_OFF_B_UB = 2 * NU + 2 * NI
_NTOT = 2 * NU + 2 * NI + NB
_NTOTP = 321536


def _scale_body(f, w, o):
    o[...] = f[...] * w[...]


def _scale(feat, w):
    n = feat.shape[0]
    return pl.pallas_call(
        _scale_body,
        grid=(n // _BR,),
        in_specs=[pl.BlockSpec((_BR, D), lambda i: (i, 0)),
                  pl.BlockSpec((_BR, 1), lambda i: (i, 0))],
        out_specs=pl.BlockSpec((_BR, D), lambda i: (i, 0)),
        out_shape=jax.ShapeDtypeStruct((n, D), jnp.float32),
    )(feat, w)


def _post_body(raw, w, f0, x1, acc):
    h = w[...] * raw[...]
    s = jnp.sum(h * h, axis=1, keepdims=True)
    nrm = h / jnp.maximum(jnp.sqrt(s), 1e-12)
    x1[...] = w[...] * h
    acc[...] = f0[...] + nrm


def _post(raw, w, f0):
    n = raw.shape[0]
    return pl.pallas_call(
        _post_body,
        grid=(n // _BR,),
        in_specs=[pl.BlockSpec((_BR, D), lambda i: (i, 0)),
                  pl.BlockSpec((_BR, 1), lambda i: (i, 0)),
                  pl.BlockSpec((_BR, D), lambda i: (i, 0))],
        out_specs=(pl.BlockSpec((_BR, D), lambda i: (i, 0)),
                   pl.BlockSpec((_BR, D), lambda i: (i, 0))),
        out_shape=(jax.ShapeDtypeStruct((n, D), jnp.float32),
                   jax.ShapeDtypeStruct((n, D), jnp.float32)),
    )(raw, w, f0)


def _final_body(raw, w, acc, o):
    h = w[...] * raw[...]
    s = jnp.sum(h * h, axis=1, keepdims=True)
    nrm = h / jnp.maximum(jnp.sqrt(s), 1e-12)
    o[...] = (acc[...] + nrm) * (1.0 / 3.0)


def _final(raw, w, acc):
    n = raw.shape[0]
    return pl.pallas_call(
        _final_body,
        grid=(n // _BR,),
        in_specs=[pl.BlockSpec((_BR, D), lambda i: (i, 0)),
                  pl.BlockSpec((_BR, 1), lambda i: (i, 0)),
                  pl.BlockSpec((_BR, D), lambda i: (i, 0))],
        out_specs=pl.BlockSpec((_BR, D), lambda i: (i, 0)),
        out_shape=jax.ShapeDtypeStruct((n, D), jnp.float32),
    )(raw, w, acc)


def _wfin_body(p, o):
    x = p[...]
    deg = x[0:1, :] + x[1:2, :]
    w = 1.0 / (jnp.sqrt(deg) + 1e-8)
    o[...] = jnp.concatenate([w, deg], axis=0)


def _wfin(partials):
    blc = 2048
    return pl.pallas_call(
        _wfin_body,
        grid=(_NTOTP // blc,),
        in_specs=[pl.BlockSpec((2, blc), lambda i: (0, i))],
        out_specs=pl.BlockSpec((2, blc), lambda i: (0, i)),
        out_shape=jax.ShapeDtypeStruct((2, _NTOTP), jnp.float32),
    )(partials)


def _enrich_body(agg, d0, d1, wt, o):
    pe = jnp.log(1.0 + d0[...]) * wt[0:1, :] + jnp.log(1.0 + d1[...]) * wt[1:2, :]
    o[...] = agg[...] + W_DEG * pe


def _enrich(agg, deg_bi, deg_ui, wt):
    n = agg.shape[0]
    return pl.pallas_call(
        _enrich_body,
        grid=(n // _BR,),
        in_specs=[pl.BlockSpec((_BR, D), lambda i: (i, 0)),
                  pl.BlockSpec((_BR, 1), lambda i: (i, 0)),
                  pl.BlockSpec((_BR, 1), lambda i: (i, 0)),
                  pl.BlockSpec((2, D), lambda i: (0, 0))],
        out_specs=pl.BlockSpec((_BR, D), lambda i: (i, 0)),
        out_shape=jax.ShapeDtypeStruct((n, D), jnp.float32),
    )(agg, deg_bi, deg_ui, wt)


def _hist(idx2d):
    totp = idx2d.shape[0] * 128
    ept = totp // (NCORES * NTILES)
    nwin = ept // _WH
    sl = _NTOTP // NTILES
    zb = 2048
    mesh = plsc.VectorSubcoreMesh(core_axis_name="c", subcore_axis_name="s")

    @functools.partial(
        pl.kernel, mesh=mesh,
        out_type=jax.ShapeDtypeStruct((NCORES, _NTOTP), jnp.float32),
        scratch_types=[pltpu.VMEM((16, 128), jnp.int32),
                       pltpu.VMEM((128,), jnp.float32),
                       pltpu.VMEM((zb,), jnp.float32),
                       pltpu.VMEM_SHARED((_NTOTP,), jnp.float32),
                       pltpu.SemaphoreType.DMA],
        compiler_params=pltpu.CompilerParams(use_tc_tiling_on_sc=False),
        name="sc_hist")
    def k(idx_hbm, out_hbm, idx_v, ones_v, z_v, hist_sh, sem):
        c = lax.axis_index("c")
        s = lax.axis_index("s")

        def fill1(i, _):
            ones_v[pl.ds(i * 16, 16)] = jnp.full((16,), 1.0, jnp.float32)
            return 0
        lax.fori_loop(0, 128 // 16, fill1, 0)

        def fillz(i, _):
            z_v[pl.ds(i * 16, 16)] = jnp.zeros((16,), jnp.float32)
            return 0
        lax.fori_loop(0, zb // 16, fillz, 0)

        nz, rem = divmod(sl, zb)
        for t in range(nz):
            pltpu.sync_copy(z_v, hist_sh.at[pl.ds(s * sl + t * zb, zb)])
        if rem:
            pltpu.sync_copy(z_v.at[pl.ds(0, rem)],
                            hist_sh.at[pl.ds(s * sl + nz * zb, rem)])
        plsc.subcore_barrier()

        rowbase = (c * NTILES + s) * (ept // 128)

        def win(wi, _):
            r0 = rowbase + wi * (_WH // 128)
            pltpu.sync_copy(idx_hbm.at[pl.ds(r0, _WH // 128)], idx_v)
            for j in range(_WH // 128):
                pltpu.sync_copy(ones_v, hist_sh.at[idx_v.at[j]], add=True)
            return 0
        lax.fori_loop(0, nwin, win, 0)

        plsc.subcore_barrier()
        pltpu.sync_copy(hist_sh.at[pl.ds(s * sl, sl)],
                        out_hbm.at[c, pl.ds(s * sl, sl)])

    return k(idx2d)


def _spmm(x, es2d, ed2d, n_chunks, chunk):
    ep = es2d.shape[0] * 128
    ept = ep // NTILES
    nwin = ept // _WE
    cp = chunk + 128
    ct = cp // NTILES
    cw = chunk // NTILES
    passes = n_chunks // NCORES
    zr = 64
    mesh = plsc.VectorSubcoreMesh(core_axis_name="c", subcore_axis_name="s")

    @functools.partial(
        pl.kernel, mesh=mesh,
        out_type=jax.ShapeDtypeStruct((n_chunks * chunk, D), jnp.float32),
        scratch_types=[pltpu.VMEM((_WE // 128, 128), jnp.int32),
                       pltpu.VMEM((_WE // 128, 128), jnp.int32),
                       pltpu.VMEM((_WE // 128, 128), jnp.int32),
                       pltpu.VMEM((_WE // 128, 128, D), jnp.float32),
                       pltpu.VMEM((zr, D), jnp.float32),
                       pltpu.VMEM_SHARED((cp, D), jnp.float32),
                       pltpu.SemaphoreType.DMA],
        compiler_params=pltpu.CompilerParams(use_tc_tiling_on_sc=False),
        name="sc_spmm")
    def k(x_hbm, es_hbm, ed_hbm, out_hbm, es_v, ed_v, rel_v, rows_v, zer_v,
          acc_sh, sem):
        c = lax.axis_index("c")
        s = lax.axis_index("s")

        def fz(i, _):
            zer_v[i // 4, pl.ds((i % 4) * 16, 16)] = jnp.zeros((16,), jnp.float32)
            return 0
        lax.fori_loop(0, zr * 4, fz, 0)

        for p in range(passes):
            ck = p * NCORES + c
            base = ck * chunk

            nz, rem = divmod(ct, zr)
            for t in range(nz):
                pltpu.sync_copy(zer_v, acc_sh.at[pl.ds(s * ct + t * zr, zr)])
            if rem:
                pltpu.sync_copy(zer_v.at[pl.ds(0, rem)],
                                acc_sh.at[pl.ds(s * ct + nz * zr, rem)])
            plsc.subcore_barrier()

            def win(wi, _):
                r0 = s * (ept // 128) + wi * (_WE // 128)
                pltpu.sync_copy(es_hbm.at[pl.ds(r0, _WE // 128)], es_v)
                pltpu.sync_copy(ed_hbm.at[pl.ds(r0, _WE // 128)], ed_v)
                cps = [pltpu.async_copy(x_hbm.at[es_v.at[j]], rows_v.at[j], sem)
                       for j in range(_WE // 128)]

                def reb(q, _):
                    v = ed_v[q // 8, pl.ds((q % 8) * 16, 16)]
                    rel = v - base
                    ok = (rel >= 0) & (rel < chunk)
                    dum = chunk + (v & 127)
                    rel_v[q // 8, pl.ds((q % 8) * 16, 16)] = jnp.where(ok, rel, dum)
                    return 0
                lax.fori_loop(0, _WE // 16, reb, 0)

                for cpd in cps:
                    cpd.wait()
                for j in range(_WE // 128):
                    pltpu.sync_copy(rows_v.at[j], acc_sh.at[rel_v.at[j]], add=True)
                return 0
            lax.fori_loop(0, nwin, win, 0)

            plsc.subcore_barrier()
            pltpu.sync_copy(acc_sh.at[pl.ds(s * cw, cw)],
                            out_hbm.at[pl.ds(base + s * cw, cw)])
            plsc.subcore_barrier()

    return k(x, es2d, ed2d)


def _bmean(enriched, bi2d, sz_pad, table):
    nbp = sz_pad.shape[0]
    bpt = nbp // (NCORES * NTILES)
    nwin = bpt // _WB
    nsub = (_WB * NT) // 128
    mesh = plsc.VectorSubcoreMesh(core_axis_name="c", subcore_axis_name="s")

    @functools.partial(
        pl.kernel, mesh=mesh,
        out_type=jax.ShapeDtypeStruct((nbp, D), jnp.float32),
        scratch_types=[pltpu.VMEM((nsub, 128), jnp.int32),
                       pltpu.VMEM((_WB,), jnp.int32),
                       pltpu.VMEM((nsub, 128, D), jnp.float32),
                       pltpu.VMEM((_WB, D), jnp.float32),
                       pltpu.VMEM((_WB, D), jnp.float32),
                       pltpu.SemaphoreType.DMA],
        compiler_params=pltpu.CompilerParams(use_tc_tiling_on_sc=False),
        name="sc_bmean")
    def k(enr_hbm, bi_hbm, sz_hbm, tab_hbm, out_hbm, ii_v, sz_v, rows_v,
          szr_v, acc_v, sem):
        c = lax.axis_index("c")
        s = lax.axis_index("s")
        wid = s * NCORES + c
        b0 = wid * bpt

        def win(wi, _):
            b = b0 + wi * _WB
            r0 = (b * NT) // 128
            pltpu.sync_copy(bi_hbm.at[pl.ds(r0, nsub)], ii_v)
            pltpu.sync_copy(sz_hbm.at[pl.ds(b, _WB)], sz_v)
            cps = [pltpu.async_copy(enr_hbm.at[ii_v.at[j]], rows_v.at[j], sem)
                   for j in range(nsub)]
            cp2 = pltpu.async_copy(tab_hbm.at[sz_v], szr_v, sem)

            def fz(i, _):
                acc_v[i // 4, pl.ds((i % 4) * 16, 16)] = jnp.zeros((16,), jnp.float32)
                return 0
            lax.fori_loop(0, _WB * 4, fz, 0)

            for cpd in cps:
                cpd.wait()
            cp2.wait()

            def facc(r, _):
                j = r // 128
                kk = r % 128
                bb = r // NT

                def fd(d, _):
                    cur = acc_v[bb, pl.ds(d * 16, 16)]
                    acc_v[bb, pl.ds(d * 16, 16)] = cur + rows_v[j, kk, pl.ds(d * 16, 16)]
                    return 0
                lax.fori_loop(0, 4, fd, 0)
                return 0
            lax.fori_loop(0, _WB * NT, facc, 0)

            def ffin(i, _):
                bb = i // 4
                d = i % 4
                acc_v[bb, pl.ds(d * 16, 16)] = (
                    acc_v[bb, pl.ds(d * 16, 16)] * (1.0 / NT)
                    + szr_v[bb, pl.ds(d * 16, 16)] * W_SIZE)
                return 0
            lax.fori_loop(0, _WB * 4, ffin, 0)

            pltpu.sync_copy(acc_v, out_hbm.at[pl.ds(b, _WB)])
            return 0
        lax.fori_loop(0, nwin, win, 0)

    return k(enriched, bi2d, sz_pad, table)


def _chunk_cfg(n):
    passes = -(-n // (NCORES * 25088))
    nc = NCORES * passes
    chunk = -(-n // (nc * 128)) * 128
    return nc, chunk


def _pad_edges(src, dst):
    e = src.shape[0]
    ep = -(-e // 16384) * 16384
    src_p = jnp.concatenate([src.astype(jnp.int32),
                             jnp.zeros((ep - e,), jnp.int32)])
    dst_p = jnp.concatenate([dst.astype(jnp.int32),
                             jnp.full((ep - e,), -1, jnp.int32)])
    return src_p.reshape(ep // 128, 128), dst_p.reshape(ep // 128, 128)


def _run_graph(feat_a, feat_b, w_a, w_b, e_src_b, e_dst_a, e_src_a, e_dst_b):
    na, nb_ = feat_a.shape[0], feat_b.shape[0]
    nca, ca = _chunk_cfg(na)
    ncb, cb = _chunk_cfg(nb_)
    xa0 = _scale(feat_a, w_a)
    xb0 = _scale(feat_b, w_b)
    ra1 = _spmm(xb0, e_src_b, e_dst_a, nca, ca)[:na]
    rb1 = _spmm(xa0, e_src_a, e_dst_b, ncb, cb)[:nb_]
    xa1, acc_a = _post(ra1, w_a, feat_a)
    xb1, acc_b = _post(rb1, w_b, feat_b)
    ra2 = _spmm(xb1, e_src_b, e_dst_a, nca, ca)[:na]
    rb2 = _spmm(xa1, e_src_a, e_dst_b, ncb, cb)[:nb_]
    out_a = _final(ra2, w_a, acc_a)
    out_b = _final(rb2, w_b, acc_b)
    return out_a, out_b


def kernel(users_feature, items_feature, bundles_feature, deg_proj_W,
           size_pe_table, ui_edges, ub_edges, bundle_items, bundle_size):
    ui = ui_edges.astype(jnp.int32)
    ub = ub_edges.astype(jnp.int32)
    bitems = bundle_items.astype(jnp.int32)
    bi_flat = bitems.reshape(-1)
    b_idx = jnp.repeat(jnp.arange(NB, dtype=jnp.int32), NT)

    idx_all = jnp.concatenate([
        ui[0] + _OFF_U_UI,
        ui[1] + _OFF_I_UI,
        bi_flat + _OFF_I_BI,
        ub[0] + _OFF_U_UB,
        ub[1] + _OFF_B_UB,
    ])
    tot = idx_all.shape[0]
    totp = -(-tot // 65536) * 65536
    pad_idx = _NTOT + (jnp.arange(totp - tot, dtype=jnp.int32) % (_NTOTP - _NTOT))
    idx2d = jnp.concatenate([idx_all, pad_idx]).reshape(totp // 128, 128)

    wd = _wfin(_hist(idx2d))
    w_full = wd[0]
    deg_full = wd[1]
    w_u_ui = w_full[_OFF_U_UI:_OFF_U_UI + NU, None]
    w_i_ui = w_full[_OFF_I_UI:_OFF_I_UI + NI, None]
    w_i_bi = w_full[_OFF_I_BI:_OFF_I_BI + NI, None]
    w_u_ub = w_full[_OFF_U_UB:_OFF_U_UB + NU, None]
    w_b_ub = w_full[_OFF_B_UB:_OFF_B_UB + NB, None]
    w_b_bi = jnp.full((NB, 1), 1.0 / (math.sqrt(NT) + 1e-8), jnp.float32)
    deg_bi = deg_full[_OFF_I_BI:_OFF_I_BI + NI, None]
    deg_ui = deg_full[_OFF_I_UI:_OFF_I_UI + NI, None]

    ui_src_i, ui_dst_u = _pad_edges(ui[1], ui[0])
    ui_src_u, ui_dst_i = _pad_edges(ui[0], ui[1])
    bi_src_i, bi_dst_b = _pad_edges(bi_flat, b_idx)
    bi_src_b, bi_dst_i = _pad_edges(b_idx, bi_flat)
    ub_src_b, ub_dst_u = _pad_edges(ub[1], ub[0])
    ub_src_u, ub_dst_b = _pad_edges(ub[0], ub[1])

    ui_u, ui_i = _run_graph(users_feature, items_feature, w_u_ui, w_i_ui,
                            ui_src_i, ui_dst_u, ui_src_u, ui_dst_i)
    bi_b, bi_i = _run_graph(bundles_feature, items_feature, w_b_bi, w_i_bi,
                            bi_src_i, bi_dst_b, bi_src_b, bi_dst_i)
    ub_u, ub_b = _run_graph(users_feature, bundles_feature, w_u_ub, w_b_ub,
                            ub_src_b, ub_dst_u, ub_src_u, ub_dst_b)

    enriched = _enrich(ui_i, deg_bi, deg_ui, deg_proj_W.T)

    nbp = -(-NB // (NCORES * NTILES * _WB)) * (NCORES * NTILES * _WB)
    bi2d = jnp.concatenate([bi_flat,
                            jnp.zeros((nbp * NT - NB * NT,), jnp.int32)
                            ]).reshape((nbp * NT) // 128, 128)
    sz_idx = jnp.clip(bundle_size, 0, NT).astype(jnp.int32)
    sz_pad = jnp.concatenate([sz_idx, jnp.zeros((nbp - NB,), jnp.int32)])
    bundle_emb = _bmean(enriched, bi2d, sz_pad, size_pe_table)[:NB]

    return (ui_u, ub_u, bi_b, bi_i, ub_b, ui_i, bundle_emb)


# speedup vs baseline: 14.7341x; 1.6283x over previous
import functools
import math

import jax
import jax.numpy as jnp
from jax import lax
from jax.experimental import pallas as pl
from jax.experimental.pallas import tpu as pltpu
from jax.experimental.pallas import tpu_sc as plsc

NU = 100000
NI = 50000
NB = 20000
D = 64
NT = 20
W_DEG = 0.05
W_SIZE = 0.05

NCORES = 2
NTILES = 16
_BR = 400
_WH = 2048
_WE = 256
_WB = 32

_OFF_U_UI = 0
_OFF_I_UI = NU
_OFF_I_BI = NU + NI
_OFF_U_UB = NU + 2 * NI
_OFF_B_UB = 2 * NU + 2 * NI
_NTOT = 2 * NU + 2 * NI + NB
_NTOTP = 321536


def _scale_body(f, w, o):
    o[...] = f[...] * w[...]


def _scale(feat, w):
    n = feat.shape[0]
    return pl.pallas_call(
        _scale_body,
        grid=(n // _BR,),
        in_specs=[pl.BlockSpec((_BR, D), lambda i: (i, 0)),
                  pl.BlockSpec((_BR, 1), lambda i: (i, 0))],
        out_specs=pl.BlockSpec((_BR, D), lambda i: (i, 0)),
        out_shape=jax.ShapeDtypeStruct((n, D), jnp.float32),
    )(feat, w)


def _post_body(raw, w, f0, x1, acc):
    h = w[...] * raw[...]
    s = jnp.sum(h * h, axis=1, keepdims=True)
    nrm = h / jnp.maximum(jnp.sqrt(s), 1e-12)
    x1[...] = w[...] * h
    acc[...] = f0[...] + nrm


def _post(raw, w, f0):
    n = raw.shape[0]
    return pl.pallas_call(
        _post_body,
        grid=(n // _BR,),
        in_specs=[pl.BlockSpec((_BR, D), lambda i: (i, 0)),
                  pl.BlockSpec((_BR, 1), lambda i: (i, 0)),
                  pl.BlockSpec((_BR, D), lambda i: (i, 0))],
        out_specs=(pl.BlockSpec((_BR, D), lambda i: (i, 0)),
                   pl.BlockSpec((_BR, D), lambda i: (i, 0))),
        out_shape=(jax.ShapeDtypeStruct((n, D), jnp.float32),
                   jax.ShapeDtypeStruct((n, D), jnp.float32)),
    )(raw, w, f0)


def _final_body(raw, w, acc, o):
    h = w[...] * raw[...]
    s = jnp.sum(h * h, axis=1, keepdims=True)
    nrm = h / jnp.maximum(jnp.sqrt(s), 1e-12)
    o[...] = (acc[...] + nrm) * (1.0 / 3.0)


def _final(raw, w, acc):
    n = raw.shape[0]
    return pl.pallas_call(
        _final_body,
        grid=(n // _BR,),
        in_specs=[pl.BlockSpec((_BR, D), lambda i: (i, 0)),
                  pl.BlockSpec((_BR, 1), lambda i: (i, 0)),
                  pl.BlockSpec((_BR, D), lambda i: (i, 0))],
        out_specs=pl.BlockSpec((_BR, D), lambda i: (i, 0)),
        out_shape=jax.ShapeDtypeStruct((n, D), jnp.float32),
    )(raw, w, acc)


def _wfin_body(p, o):
    x = p[...]
    deg = x[0:1, :] + x[1:2, :]
    w = 1.0 / (jnp.sqrt(deg) + 1e-8)
    o[...] = jnp.concatenate([w, deg], axis=0)


def _wfin(partials):
    blc = 2048
    return pl.pallas_call(
        _wfin_body,
        grid=(_NTOTP // blc,),
        in_specs=[pl.BlockSpec((2, blc), lambda i: (0, i))],
        out_specs=pl.BlockSpec((2, blc), lambda i: (0, i)),
        out_shape=jax.ShapeDtypeStruct((2, _NTOTP), jnp.float32),
    )(partials)


def _enrich_body(agg, d0, d1, wt, o):
    pe = jnp.log(1.0 + d0[...]) * wt[0:1, :] + jnp.log(1.0 + d1[...]) * wt[1:2, :]
    o[...] = agg[...] + W_DEG * pe


def _enrich(agg, deg_bi, deg_ui, wt):
    n = agg.shape[0]
    return pl.pallas_call(
        _enrich_body,
        grid=(n // _BR,),
        in_specs=[pl.BlockSpec((_BR, D), lambda i: (i, 0)),
                  pl.BlockSpec((_BR, 1), lambda i: (i, 0)),
                  pl.BlockSpec((_BR, 1), lambda i: (i, 0)),
                  pl.BlockSpec((2, D), lambda i: (0, 0))],
        out_specs=pl.BlockSpec((_BR, D), lambda i: (i, 0)),
        out_shape=jax.ShapeDtypeStruct((n, D), jnp.float32),
    )(agg, deg_bi, deg_ui, wt)


def _hist(idx2d):
    totp = idx2d.shape[0] * 128
    ept = totp // (NCORES * NTILES)
    nwin = ept // _WH
    sl = _NTOTP // NTILES
    zb = 2048
    mesh = plsc.VectorSubcoreMesh(core_axis_name="c", subcore_axis_name="s")

    @functools.partial(
        pl.kernel, mesh=mesh,
        out_type=jax.ShapeDtypeStruct((NCORES, _NTOTP), jnp.float32),
        scratch_types=[pltpu.VMEM((16, 128), jnp.int32),
                       pltpu.VMEM((128,), jnp.float32),
                       pltpu.VMEM((zb,), jnp.float32),
                       pltpu.VMEM_SHARED((_NTOTP,), jnp.float32),
                       pltpu.SemaphoreType.DMA],
        compiler_params=pltpu.CompilerParams(use_tc_tiling_on_sc=False),
        name="sc_hist")
    def k(idx_hbm, out_hbm, idx_v, ones_v, z_v, hist_sh, sem):
        c = lax.axis_index("c")
        s = lax.axis_index("s")

        def fill1(i, _):
            ones_v[pl.ds(i * 16, 16)] = jnp.full((16,), 1.0, jnp.float32)
            return 0
        lax.fori_loop(0, 128 // 16, fill1, 0)

        def fillz(i, _):
            z_v[pl.ds(i * 16, 16)] = jnp.zeros((16,), jnp.float32)
            return 0
        lax.fori_loop(0, zb // 16, fillz, 0)

        nz, rem = divmod(sl, zb)
        for t in range(nz):
            pltpu.sync_copy(z_v, hist_sh.at[pl.ds(s * sl + t * zb, zb)])
        if rem:
            pltpu.sync_copy(z_v.at[pl.ds(0, rem)],
                            hist_sh.at[pl.ds(s * sl + nz * zb, rem)])
        plsc.subcore_barrier()

        rowbase = (c * NTILES + s) * (ept // 128)

        def win(wi, _):
            r0 = rowbase + wi * (_WH // 128)
            pltpu.sync_copy(idx_hbm.at[pl.ds(r0, _WH // 128)], idx_v)
            for j in range(_WH // 128):
                pltpu.sync_copy(ones_v, hist_sh.at[idx_v.at[j]], add=True)
            return 0
        lax.fori_loop(0, nwin, win, 0)

        plsc.subcore_barrier()
        pltpu.sync_copy(hist_sh.at[pl.ds(s * sl, sl)],
                        out_hbm.at[c, pl.ds(s * sl, sl)])

    return k(idx2d)


def _spmm(x, es2d, ed2d, n_chunks, chunk):
    ep = es2d.shape[0] * 128
    ept = ep // NTILES
    nwin = ept // _WE
    cp = chunk + 128
    ct = cp // NTILES
    cw = chunk // NTILES
    passes = n_chunks // NCORES
    zr = 64
    cap = _WE + 256
    mesh = plsc.VectorSubcoreMesh(core_axis_name="c", subcore_axis_name="s")

    @functools.partial(
        pl.kernel, mesh=mesh,
        out_type=jax.ShapeDtypeStruct((n_chunks * chunk, D), jnp.float32),
        scratch_types=[pltpu.VMEM((_WE // 128, 128), jnp.int32),
                       pltpu.VMEM((_WE // 128, 128), jnp.int32),
                       pltpu.VMEM((cap,), jnp.int32),
                       pltpu.VMEM((cap,), jnp.int32),
                       pltpu.VMEM((128,), jnp.int32),
                       pltpu.VMEM((128,), jnp.int32),
                       pltpu.VMEM((128, D), jnp.float32),
                       pltpu.VMEM((zr, D), jnp.float32),
                       pltpu.VMEM_SHARED((cp, D), jnp.float32),
                       pltpu.SemaphoreType.DMA],
        compiler_params=pltpu.CompilerParams(use_tc_tiling_on_sc=False,
                                             needs_layout_passes=False),
        name="sc_spmm")
    def k(x_hbm, es_hbm, ed_hbm, out_hbm, es_v, ed_v, csrc, crel, s128, r128,
          rows_v, zer_v, acc_sh, sem):
        c = lax.axis_index("c")
        s = lax.axis_index("s")
        lane = lax.iota(jnp.int32, 16)
        one_v = jnp.ones((16,), jnp.int32)
        zero_v = jnp.zeros((16,), jnp.int32)
        last_v = jnp.full((16,), 15, jnp.int32)

        def fz(i, _):
            zer_v[i // 4, pl.ds((i % 4) * 16, 16)] = jnp.zeros((16,), jnp.float32)
            return 0
        lax.fori_loop(0, zr * 4, fz, 0)

        for p in range(passes):
            ck = p * NCORES + c
            base = ck * chunk

            nz, rem = divmod(ct, zr)
            for t in range(nz):
                pltpu.sync_copy(zer_v, acc_sh.at[pl.ds(s * ct + t * zr, zr)])
            if rem:
                pltpu.sync_copy(zer_v.at[pl.ds(0, rem)],
                                acc_sh.at[pl.ds(s * ct + nz * zr, rem)])
            plsc.subcore_barrier()

            def win(wi, _):
                r0 = s * (ept // 128) + wi * (_WE // 128)
                pltpu.sync_copy(es_hbm.at[pl.ds(r0, _WE // 128)], es_v)
                pltpu.sync_copy(ed_hbm.at[pl.ds(r0, _WE // 128)], ed_v)

                def comp(q, run_vec):
                    v = ed_v[q // 8, pl.ds((q % 8) * 16, 16)]
                    sv = es_v[q // 8, pl.ds((q % 8) * 16, 16)]
                    rel = v - base
                    ok = (rel >= 0) & (rel < chunk)
                    cs = plsc.cumsum(jnp.where(ok, one_v, zero_v))
                    pos = run_vec + cs - 1
                    plsc.store_scatter(crel, [pos], rel, mask=ok)
                    plsc.store_scatter(csrc, [pos], sv, mask=ok)
                    tot = lax.gather(
                        cs, last_v[:, None],
                        lax.GatherDimensionNumbers(offset_dims=(),
                                                   collapsed_slice_dims=(0,),
                                                   start_index_map=(0,)),
                        slice_sizes=(1,),
                        mode=lax.GatherScatterMode.PROMISE_IN_BOUNDS)
                    return run_vec + tot
                run_vec = lax.fori_loop(0, _WE // 16, comp, zero_v)

                for kk in range(8):
                    plsc.store_scatter(crel, [run_vec + kk * 16 + lane],
                                       chunk + kk * 16 + lane)
                    plsc.store_scatter(csrc, [run_vec + kk * 16 + lane],
                                       kk * 16 + lane)
                nsub_vec = (run_vec + 127) // 128

                def sub_cond(sb):
                    return jnp.any(nsub_vec > sb)

                def sub(sb):
                    for t in range(8):
                        s128[pl.ds(t * 16, 16)] = csrc[pl.ds(sb * 128 + t * 16, 16)]
                        r128[pl.ds(t * 16, 16)] = crel[pl.ds(sb * 128 + t * 16, 16)]
                    pltpu.async_copy(x_hbm.at[s128], rows_v, sem).wait()
                    pltpu.sync_copy(rows_v, acc_sh.at[r128], add=True)
                    return sb + 1
                lax.while_loop(sub_cond, sub, 0)
                return 0
            lax.fori_loop(0, nwin, win, 0)

            plsc.subcore_barrier()
            pltpu.sync_copy(acc_sh.at[pl.ds(s * cw, cw)],
                            out_hbm.at[pl.ds(base + s * cw, cw)])
            plsc.subcore_barrier()

    return k(x, es2d, ed2d)


def _bmean(enriched, bi2d, sz_pad, table):
    nbp = sz_pad.shape[0]
    bpt = nbp // (NCORES * NTILES)
    nwin = bpt // _WB
    nsub = (_WB * NT) // 128
    mesh = plsc.VectorSubcoreMesh(core_axis_name="c", subcore_axis_name="s")

    @functools.partial(
        pl.kernel, mesh=mesh,
        out_type=jax.ShapeDtypeStruct((nbp, D), jnp.float32),
        scratch_types=[pltpu.VMEM((nsub, 128), jnp.int32),
                       pltpu.VMEM((_WB,), jnp.int32),
                       pltpu.VMEM((nsub, 128, D), jnp.float32),
                       pltpu.VMEM((_WB, D), jnp.float32),
                       pltpu.VMEM((_WB, D), jnp.float32),
                       pltpu.SemaphoreType.DMA],
        compiler_params=pltpu.CompilerParams(use_tc_tiling_on_sc=False),
        name="sc_bmean")
    def k(enr_hbm, bi_hbm, sz_hbm, tab_hbm, out_hbm, ii_v, sz_v, rows_v,
          szr_v, acc_v, sem):
        c = lax.axis_index("c")
        s = lax.axis_index("s")
        wid = s * NCORES + c
        b0 = wid * bpt

        def win(wi, _):
            b = b0 + wi * _WB
            r0 = (b * NT) // 128
            pltpu.sync_copy(bi_hbm.at[pl.ds(r0, nsub)], ii_v)
            pltpu.sync_copy(sz_hbm.at[pl.ds(b, _WB)], sz_v)
            cps = [pltpu.async_copy(enr_hbm.at[ii_v.at[j]], rows_v.at[j], sem)
                   for j in range(nsub)]
            cp2 = pltpu.async_copy(tab_hbm.at[sz_v], szr_v, sem)

            def fz(i, _):
                acc_v[i // 4, pl.ds((i % 4) * 16, 16)] = jnp.zeros((16,), jnp.float32)
                return 0
            lax.fori_loop(0, _WB * 4, fz, 0)

            for cpd in cps:
                cpd.wait()
            cp2.wait()

            def facc(r, _):
                j = r // 128
                kk = r % 128
                bb = r // NT

                def fd(d, _):
                    cur = acc_v[bb, pl.ds(d * 16, 16)]
                    acc_v[bb, pl.ds(d * 16, 16)] = cur + rows_v[j, kk, pl.ds(d * 16, 16)]
                    return 0
                lax.fori_loop(0, 4, fd, 0)
                return 0
            lax.fori_loop(0, _WB * NT, facc, 0)

            def ffin(i, _):
                bb = i // 4
                d = i % 4
                acc_v[bb, pl.ds(d * 16, 16)] = (
                    acc_v[bb, pl.ds(d * 16, 16)] * (1.0 / NT)
                    + szr_v[bb, pl.ds(d * 16, 16)] * W_SIZE)
                return 0
            lax.fori_loop(0, _WB * 4, ffin, 0)

            pltpu.sync_copy(acc_v, out_hbm.at[pl.ds(b, _WB)])
            return 0
        lax.fori_loop(0, nwin, win, 0)

    return k(enriched, bi2d, sz_pad, table)


def _chunk_cfg(n):
    passes = -(-n // (NCORES * 25088))
    nc = NCORES * passes
    chunk = -(-n // (nc * 128)) * 128
    return nc, chunk


def _pad_edges(src, dst):
    e = src.shape[0]
    ep = -(-e // 16384) * 16384
    src_p = jnp.concatenate([src.astype(jnp.int32),
                             jnp.zeros((ep - e,), jnp.int32)])
    dst_p = jnp.concatenate([dst.astype(jnp.int32),
                             jnp.full((ep - e,), -1, jnp.int32)])
    return src_p.reshape(ep // 128, 128), dst_p.reshape(ep // 128, 128)


def _run_graph(feat_a, feat_b, w_a, w_b, e_src_b, e_dst_a, e_src_a, e_dst_b):
    na, nb_ = feat_a.shape[0], feat_b.shape[0]
    nca, ca = _chunk_cfg(na)
    ncb, cb = _chunk_cfg(nb_)
    xa0 = _scale(feat_a, w_a)
    xb0 = _scale(feat_b, w_b)
    ra1 = _spmm(xb0, e_src_b, e_dst_a, nca, ca)[:na]
    rb1 = _spmm(xa0, e_src_a, e_dst_b, ncb, cb)[:nb_]
    xa1, acc_a = _post(ra1, w_a, feat_a)
    xb1, acc_b = _post(rb1, w_b, feat_b)
    ra2 = _spmm(xb1, e_src_b, e_dst_a, nca, ca)[:na]
    rb2 = _spmm(xa1, e_src_a, e_dst_b, ncb, cb)[:nb_]
    out_a = _final(ra2, w_a, acc_a)
    out_b = _final(rb2, w_b, acc_b)
    return out_a, out_b


def kernel(users_feature, items_feature, bundles_feature, deg_proj_W,
           size_pe_table, ui_edges, ub_edges, bundle_items, bundle_size):
    ui = ui_edges.astype(jnp.int32)
    ub = ub_edges.astype(jnp.int32)
    bitems = bundle_items.astype(jnp.int32)
    bi_flat = bitems.reshape(-1)
    b_idx = jnp.repeat(jnp.arange(NB, dtype=jnp.int32), NT)

    idx_all = jnp.concatenate([
        ui[0] + _OFF_U_UI,
        ui[1] + _OFF_I_UI,
        bi_flat + _OFF_I_BI,
        ub[0] + _OFF_U_UB,
        ub[1] + _OFF_B_UB,
    ])
    tot = idx_all.shape[0]
    totp = -(-tot // 65536) * 65536
    pad_idx = _NTOT + (jnp.arange(totp - tot, dtype=jnp.int32) % (_NTOTP - _NTOT))
    idx2d = jnp.concatenate([idx_all, pad_idx]).reshape(totp // 128, 128)

    wd = _wfin(_hist(idx2d))
    w_full = wd[0]
    deg_full = wd[1]
    w_u_ui = w_full[_OFF_U_UI:_OFF_U_UI + NU, None]
    w_i_ui = w_full[_OFF_I_UI:_OFF_I_UI + NI, None]
    w_i_bi = w_full[_OFF_I_BI:_OFF_I_BI + NI, None]
    w_u_ub = w_full[_OFF_U_UB:_OFF_U_UB + NU, None]
    w_b_ub = w_full[_OFF_B_UB:_OFF_B_UB + NB, None]
    w_b_bi = jnp.full((NB, 1), 1.0 / (math.sqrt(NT) + 1e-8), jnp.float32)
    deg_bi = deg_full[_OFF_I_BI:_OFF_I_BI + NI, None]
    deg_ui = deg_full[_OFF_I_UI:_OFF_I_UI + NI, None]

    ui_src_i, ui_dst_u = _pad_edges(ui[1], ui[0])
    ui_src_u, ui_dst_i = _pad_edges(ui[0], ui[1])
    bi_src_i, bi_dst_b = _pad_edges(bi_flat, b_idx)
    bi_src_b, bi_dst_i = _pad_edges(b_idx, bi_flat)
    ub_src_b, ub_dst_u = _pad_edges(ub[1], ub[0])
    ub_src_u, ub_dst_b = _pad_edges(ub[0], ub[1])

    ui_u, ui_i = _run_graph(users_feature, items_feature, w_u_ui, w_i_ui,
                            ui_src_i, ui_dst_u, ui_src_u, ui_dst_i)
    bi_b, bi_i = _run_graph(bundles_feature, items_feature, w_b_bi, w_i_bi,
                            bi_src_i, bi_dst_b, bi_src_b, bi_dst_i)
    ub_u, ub_b = _run_graph(users_feature, bundles_feature, w_u_ub, w_b_ub,
                            ub_src_b, ub_dst_u, ub_src_u, ub_dst_b)

    enriched = _enrich(ui_i, deg_bi, deg_ui, deg_proj_W.T)

    nbp = -(-NB // (NCORES * NTILES * _WB)) * (NCORES * NTILES * _WB)
    bi2d = jnp.concatenate([bi_flat,
                            jnp.zeros((nbp * NT - NB * NT,), jnp.int32)
                            ]).reshape((nbp * NT) // 128, 128)
    sz_idx = jnp.clip(bundle_size, 0, NT).astype(jnp.int32)
    sz_pad = jnp.concatenate([sz_idx, jnp.zeros((nbp - NB,), jnp.int32)])
    bundle_emb = _bmean(enriched, bi2d, sz_pad, size_pe_table)[:NB]

    return (ui_u, ub_u, bi_b, bi_i, ub_b, ui_i, bundle_emb)


# speedup vs baseline: 23.7858x; 1.6143x over previous
import functools
import math

import jax
import jax.numpy as jnp
from jax import lax
from jax.experimental import pallas as pl
from jax.experimental.pallas import tpu as pltpu
from jax.experimental.pallas import tpu_sc as plsc

NU = 100000
NI = 50000
NB = 20000
D = 64
NT = 20
W_DEG = 0.05
W_SIZE = 0.05

NCORES = 2
NTILES = 16
_BR = 400
_WH = 2048
_WE = 2048
_WB = 32

_OFF_U_UI = 0
_OFF_I_UI = NU
_OFF_I_BI = NU + NI
_OFF_U_UB = NU + 2 * NI
_OFF_B_UB = 2 * NU + 2 * NI
_NTOT = 2 * NU + 2 * NI + NB
_NTOTP = 321536


def _scale_body(f, w, o):
    o[...] = f[...] * w[...]


def _scale(feat, w):
    n = feat.shape[0]
    return pl.pallas_call(
        _scale_body,
        grid=(n // _BR,),
        in_specs=[pl.BlockSpec((_BR, D), lambda i: (i, 0)),
                  pl.BlockSpec((_BR, 1), lambda i: (i, 0))],
        out_specs=pl.BlockSpec((_BR, D), lambda i: (i, 0)),
        out_shape=jax.ShapeDtypeStruct((n, D), jnp.float32),
    )(feat, w)


def _post_body(raw, w, f0, x1, acc):
    h = w[...] * raw[...]
    s = jnp.sum(h * h, axis=1, keepdims=True)
    nrm = h / jnp.maximum(jnp.sqrt(s), 1e-12)
    x1[...] = w[...] * h
    acc[...] = f0[...] + nrm


def _post(raw, w, f0):
    n = raw.shape[0]
    return pl.pallas_call(
        _post_body,
        grid=(n // _BR,),
        in_specs=[pl.BlockSpec((_BR, D), lambda i: (i, 0)),
                  pl.BlockSpec((_BR, 1), lambda i: (i, 0)),
                  pl.BlockSpec((_BR, D), lambda i: (i, 0))],
        out_specs=(pl.BlockSpec((_BR, D), lambda i: (i, 0)),
                   pl.BlockSpec((_BR, D), lambda i: (i, 0))),
        out_shape=(jax.ShapeDtypeStruct((n, D), jnp.float32),
                   jax.ShapeDtypeStruct((n, D), jnp.float32)),
    )(raw, w, f0)


def _final_body(raw, w, acc, o):
    h = w[...] * raw[...]
    s = jnp.sum(h * h, axis=1, keepdims=True)
    nrm = h / jnp.maximum(jnp.sqrt(s), 1e-12)
    o[...] = (acc[...] + nrm) * (1.0 / 3.0)


def _final(raw, w, acc):
    n = raw.shape[0]
    return pl.pallas_call(
        _final_body,
        grid=(n // _BR,),
        in_specs=[pl.BlockSpec((_BR, D), lambda i: (i, 0)),
                  pl.BlockSpec((_BR, 1), lambda i: (i, 0)),
                  pl.BlockSpec((_BR, D), lambda i: (i, 0))],
        out_specs=pl.BlockSpec((_BR, D), lambda i: (i, 0)),
        out_shape=jax.ShapeDtypeStruct((n, D), jnp.float32),
    )(raw, w, acc)


def _wfin_body(p, o):
    x = p[...]
    deg = x[0:1, :] + x[1:2, :]
    w = 1.0 / (jnp.sqrt(deg) + 1e-8)
    o[...] = jnp.concatenate([w, deg], axis=0)


def _wfin(partials):
    blc = 2048
    return pl.pallas_call(
        _wfin_body,
        grid=(_NTOTP // blc,),
        in_specs=[pl.BlockSpec((2, blc), lambda i: (0, i))],
        out_specs=pl.BlockSpec((2, blc), lambda i: (0, i)),
        out_shape=jax.ShapeDtypeStruct((2, _NTOTP), jnp.float32),
    )(partials)


def _enrich_body(agg, d0, d1, wt, o):
    pe = jnp.log(1.0 + d0[...]) * wt[0:1, :] + jnp.log(1.0 + d1[...]) * wt[1:2, :]
    o[...] = agg[...] + W_DEG * pe


def _enrich(agg, deg_bi, deg_ui, wt):
    n = agg.shape[0]
    return pl.pallas_call(
        _enrich_body,
        grid=(n // _BR,),
        in_specs=[pl.BlockSpec((_BR, D), lambda i: (i, 0)),
                  pl.BlockSpec((_BR, 1), lambda i: (i, 0)),
                  pl.BlockSpec((_BR, 1), lambda i: (i, 0)),
                  pl.BlockSpec((2, D), lambda i: (0, 0))],
        out_specs=pl.BlockSpec((_BR, D), lambda i: (i, 0)),
        out_shape=jax.ShapeDtypeStruct((n, D), jnp.float32),
    )(agg, deg_bi, deg_ui, wt)


def _hist(idx2d):
    totp = idx2d.shape[0] * 128
    ept = totp // (NCORES * NTILES)
    nwin = ept // _WH
    sl = _NTOTP // NTILES
    zb = 2048
    mesh = plsc.VectorSubcoreMesh(core_axis_name="c", subcore_axis_name="s")

    @functools.partial(
        pl.kernel, mesh=mesh,
        out_type=jax.ShapeDtypeStruct((NCORES, _NTOTP), jnp.float32),
        scratch_types=[pltpu.VMEM((16, 128), jnp.int32),
                       pltpu.VMEM((128,), jnp.float32),
                       pltpu.VMEM((zb,), jnp.float32),
                       pltpu.VMEM_SHARED((_NTOTP,), jnp.float32),
                       pltpu.SemaphoreType.DMA],
        compiler_params=pltpu.CompilerParams(use_tc_tiling_on_sc=False),
        name="sc_hist")
    def k(idx_hbm, out_hbm, idx_v, ones_v, z_v, hist_sh, sem):
        c = lax.axis_index("c")
        s = lax.axis_index("s")

        def fill1(i, _):
            ones_v[pl.ds(i * 16, 16)] = jnp.full((16,), 1.0, jnp.float32)
            return 0
        lax.fori_loop(0, 128 // 16, fill1, 0)

        def fillz(i, _):
            z_v[pl.ds(i * 16, 16)] = jnp.zeros((16,), jnp.float32)
            return 0
        lax.fori_loop(0, zb // 16, fillz, 0)

        nz, rem = divmod(sl, zb)
        for t in range(nz):
            pltpu.sync_copy(z_v, hist_sh.at[pl.ds(s * sl + t * zb, zb)])
        if rem:
            pltpu.sync_copy(z_v.at[pl.ds(0, rem)],
                            hist_sh.at[pl.ds(s * sl + nz * zb, rem)])
        plsc.subcore_barrier()

        rowbase = (c * NTILES + s) * (ept // 128)

        def win(wi, _):
            r0 = rowbase + wi * (_WH // 128)
            pltpu.sync_copy(idx_hbm.at[pl.ds(r0, _WH // 128)], idx_v)
            for j in range(_WH // 128):
                pltpu.sync_copy(ones_v, hist_sh.at[idx_v.at[j]], add=True)
            return 0
        lax.fori_loop(0, nwin, win, 0)

        plsc.subcore_barrier()
        pltpu.sync_copy(hist_sh.at[pl.ds(s * sl, sl)],
                        out_hbm.at[c, pl.ds(s * sl, sl)])

    return k(idx2d)


def _spmm(x, es2d, ed2d, n_chunks, chunk):
    ep = es2d.shape[0] * 128
    ept = ep // NTILES
    nwin = ept // _WE
    cp = chunk + 128
    ct = cp // NTILES
    cw = chunk // NTILES
    passes = n_chunks // NCORES
    zr = 64
    cap = _WE + 256
    mesh = plsc.VectorSubcoreMesh(core_axis_name="c", subcore_axis_name="s")

    @functools.partial(
        pl.kernel, mesh=mesh,
        out_type=jax.ShapeDtypeStruct((n_chunks * chunk, D), jnp.float32),
        scratch_types=[pltpu.VMEM((_WE // 128, 128), jnp.int32),
                       pltpu.VMEM((_WE // 128, 128), jnp.int32),
                       pltpu.VMEM((cap,), jnp.int32),
                       pltpu.VMEM((cap,), jnp.int32),
                       pltpu.VMEM((128,), jnp.int32),
                       pltpu.VMEM((128,), jnp.int32),
                       pltpu.VMEM((128, D), jnp.float32),
                       pltpu.VMEM((zr, D), jnp.float32),
                       pltpu.VMEM_SHARED((cp, D), jnp.float32),
                       pltpu.SemaphoreType.DMA],
        compiler_params=pltpu.CompilerParams(use_tc_tiling_on_sc=False,
                                             needs_layout_passes=False),
        name="sc_spmm")
    def k(x_hbm, es_hbm, ed_hbm, out_hbm, es_v, ed_v, csrc, crel, s128, r128,
          rows_v, zer_v, acc_sh, sem):
        c = lax.axis_index("c")
        s = lax.axis_index("s")
        lane = lax.iota(jnp.int32, 16)
        one_v = jnp.ones((16,), jnp.int32)
        zero_v = jnp.zeros((16,), jnp.int32)
        last_v = jnp.full((16,), 15, jnp.int32)

        def fz(i, _):
            zer_v[i // 4, pl.ds((i % 4) * 16, 16)] = jnp.zeros((16,), jnp.float32)
            return 0
        lax.fori_loop(0, zr * 4, fz, 0)

        for p in range(passes):
            ck = p * NCORES + c
            base = ck * chunk

            nz, rem = divmod(ct, zr)
            for t in range(nz):
                pltpu.sync_copy(zer_v, acc_sh.at[pl.ds(s * ct + t * zr, zr)])
            if rem:
                pltpu.sync_copy(zer_v.at[pl.ds(0, rem)],
                                acc_sh.at[pl.ds(s * ct + nz * zr, rem)])
            plsc.subcore_barrier()

            def win(wi, _):
                r0 = s * (ept // 128) + wi * (_WE // 128)
                pltpu.sync_copy(es_hbm.at[pl.ds(r0, _WE // 128)], es_v)
                pltpu.sync_copy(ed_hbm.at[pl.ds(r0, _WE // 128)], ed_v)

                def comp(q, run_vec):
                    v = ed_v[q // 8, pl.ds((q % 8) * 16, 16)]
                    sv = es_v[q // 8, pl.ds((q % 8) * 16, 16)]
                    rel = v - base
                    ok = (rel >= 0) & (rel < chunk)
                    cs = plsc.cumsum(jnp.where(ok, one_v, zero_v))
                    pos = run_vec + cs - 1
                    plsc.store_scatter(crel, [pos], rel, mask=ok)
                    plsc.store_scatter(csrc, [pos], sv, mask=ok)
                    tot = lax.gather(
                        cs, last_v[:, None],
                        lax.GatherDimensionNumbers(offset_dims=(),
                                                   collapsed_slice_dims=(0,),
                                                   start_index_map=(0,)),
                        slice_sizes=(1,),
                        mode=lax.GatherScatterMode.PROMISE_IN_BOUNDS)
                    return run_vec + tot
                run_vec = lax.fori_loop(0, _WE // 16, comp, zero_v)

                for kk in range(8):
                    plsc.store_scatter(crel, [run_vec + kk * 16 + lane],
                                       chunk + kk * 16 + lane)
                    plsc.store_scatter(csrc, [run_vec + kk * 16 + lane],
                                       kk * 16 + lane)
                nsub_vec = (run_vec + 127) // 128

                def sub_cond(sb):
                    return jnp.any(nsub_vec > sb)

                def sub(sb):
                    for t in range(8):
                        s128[pl.ds(t * 16, 16)] = csrc[pl.ds(sb * 128 + t * 16, 16)]
                        r128[pl.ds(t * 16, 16)] = crel[pl.ds(sb * 128 + t * 16, 16)]
                    pltpu.async_copy(x_hbm.at[s128], rows_v, sem).wait()
                    pltpu.sync_copy(rows_v, acc_sh.at[r128], add=True)
                    return sb + 1
                lax.while_loop(sub_cond, sub, 0)
                return 0
            lax.fori_loop(0, nwin, win, 0)

            plsc.subcore_barrier()
            pltpu.sync_copy(acc_sh.at[pl.ds(s * cw, cw)],
                            out_hbm.at[pl.ds(base + s * cw, cw)])
            plsc.subcore_barrier()

    return k(x, es2d, ed2d)


def _bmean(enriched, bi2d, sz_pad, table):
    nbp = sz_pad.shape[0]
    bpt = nbp // (NCORES * NTILES)
    nwin = bpt // _WB
    nsub = (_WB * NT) // 128
    mesh = plsc.VectorSubcoreMesh(core_axis_name="c", subcore_axis_name="s")

    @functools.partial(
        pl.kernel, mesh=mesh,
        out_type=jax.ShapeDtypeStruct((nbp, D), jnp.float32),
        scratch_types=[pltpu.VMEM((nsub, 128), jnp.int32),
                       pltpu.VMEM((_WB,), jnp.int32),
                       pltpu.VMEM((nsub, 128, D), jnp.float32),
                       pltpu.VMEM((_WB, D), jnp.float32),
                       pltpu.VMEM((_WB, D), jnp.float32),
                       pltpu.SemaphoreType.DMA],
        compiler_params=pltpu.CompilerParams(use_tc_tiling_on_sc=False),
        name="sc_bmean")
    def k(enr_hbm, bi_hbm, sz_hbm, tab_hbm, out_hbm, ii_v, sz_v, rows_v,
          szr_v, acc_v, sem):
        c = lax.axis_index("c")
        s = lax.axis_index("s")
        wid = s * NCORES + c
        b0 = wid * bpt

        def win(wi, _):
            b = b0 + wi * _WB
            r0 = (b * NT) // 128
            pltpu.sync_copy(bi_hbm.at[pl.ds(r0, nsub)], ii_v)
            pltpu.sync_copy(sz_hbm.at[pl.ds(b, _WB)], sz_v)
            cps = [pltpu.async_copy(enr_hbm.at[ii_v.at[j]], rows_v.at[j], sem)
                   for j in range(nsub)]
            cp2 = pltpu.async_copy(tab_hbm.at[sz_v], szr_v, sem)

            def fz(i, _):
                acc_v[i // 4, pl.ds((i % 4) * 16, 16)] = jnp.zeros((16,), jnp.float32)
                return 0
            lax.fori_loop(0, _WB * 4, fz, 0)

            for cpd in cps:
                cpd.wait()
            cp2.wait()

            def facc(r, _):
                j = r // 128
                kk = r % 128
                bb = r // NT

                def fd(d, _):
                    cur = acc_v[bb, pl.ds(d * 16, 16)]
                    acc_v[bb, pl.ds(d * 16, 16)] = cur + rows_v[j, kk, pl.ds(d * 16, 16)]
                    return 0
                lax.fori_loop(0, 4, fd, 0)
                return 0
            lax.fori_loop(0, _WB * NT, facc, 0)

            def ffin(i, _):
                bb = i // 4
                d = i % 4
                acc_v[bb, pl.ds(d * 16, 16)] = (
                    acc_v[bb, pl.ds(d * 16, 16)] * (1.0 / NT)
                    + szr_v[bb, pl.ds(d * 16, 16)] * W_SIZE)
                return 0
            lax.fori_loop(0, _WB * 4, ffin, 0)

            pltpu.sync_copy(acc_v, out_hbm.at[pl.ds(b, _WB)])
            return 0
        lax.fori_loop(0, nwin, win, 0)

    return k(enriched, bi2d, sz_pad, table)


def _chunk_cfg(n):
    passes = -(-n // (NCORES * 25088))
    nc = NCORES * passes
    chunk = -(-n // (nc * 128)) * 128
    return nc, chunk


def _pad_edges(src, dst):
    e = src.shape[0]
    ep = -(-e // (NTILES * _WE)) * (NTILES * _WE)
    src_p = jnp.concatenate([src.astype(jnp.int32),
                             jnp.zeros((ep - e,), jnp.int32)])
    dst_p = jnp.concatenate([dst.astype(jnp.int32),
                             jnp.full((ep - e,), -1, jnp.int32)])
    return src_p.reshape(ep // 128, 128), dst_p.reshape(ep // 128, 128)


def _run_graph(feat_a, feat_b, w_a, w_b, e_src_b, e_dst_a, e_src_a, e_dst_b):
    na, nb_ = feat_a.shape[0], feat_b.shape[0]
    nca, ca = _chunk_cfg(na)
    ncb, cb = _chunk_cfg(nb_)
    xa0 = _scale(feat_a, w_a)
    xb0 = _scale(feat_b, w_b)
    ra1 = _spmm(xb0, e_src_b, e_dst_a, nca, ca)[:na]
    rb1 = _spmm(xa0, e_src_a, e_dst_b, ncb, cb)[:nb_]
    xa1, acc_a = _post(ra1, w_a, feat_a)
    xb1, acc_b = _post(rb1, w_b, feat_b)
    ra2 = _spmm(xb1, e_src_b, e_dst_a, nca, ca)[:na]
    rb2 = _spmm(xa1, e_src_a, e_dst_b, ncb, cb)[:nb_]
    out_a = _final(ra2, w_a, acc_a)
    out_b = _final(rb2, w_b, acc_b)
    return out_a, out_b


def kernel(users_feature, items_feature, bundles_feature, deg_proj_W,
           size_pe_table, ui_edges, ub_edges, bundle_items, bundle_size):
    ui = ui_edges.astype(jnp.int32)
    ub = ub_edges.astype(jnp.int32)
    bitems = bundle_items.astype(jnp.int32)
    bi_flat = bitems.reshape(-1)
    b_idx = jnp.repeat(jnp.arange(NB, dtype=jnp.int32), NT)

    idx_all = jnp.concatenate([
        ui[0] + _OFF_U_UI,
        ui[1] + _OFF_I_UI,
        bi_flat + _OFF_I_BI,
        ub[0] + _OFF_U_UB,
        ub[1] + _OFF_B_UB,
    ])
    tot = idx_all.shape[0]
    totp = -(-tot // 65536) * 65536
    pad_idx = _NTOT + (jnp.arange(totp - tot, dtype=jnp.int32) % (_NTOTP - _NTOT))
    idx2d = jnp.concatenate([idx_all, pad_idx]).reshape(totp // 128, 128)

    wd = _wfin(_hist(idx2d))
    w_full = wd[0]
    deg_full = wd[1]
    w_u_ui = w_full[_OFF_U_UI:_OFF_U_UI + NU, None]
    w_i_ui = w_full[_OFF_I_UI:_OFF_I_UI + NI, None]
    w_i_bi = w_full[_OFF_I_BI:_OFF_I_BI + NI, None]
    w_u_ub = w_full[_OFF_U_UB:_OFF_U_UB + NU, None]
    w_b_ub = w_full[_OFF_B_UB:_OFF_B_UB + NB, None]
    w_b_bi = jnp.full((NB, 1), 1.0 / (math.sqrt(NT) + 1e-8), jnp.float32)
    deg_bi = deg_full[_OFF_I_BI:_OFF_I_BI + NI, None]
    deg_ui = deg_full[_OFF_I_UI:_OFF_I_UI + NI, None]

    ui_src_i, ui_dst_u = _pad_edges(ui[1], ui[0])
    ui_src_u, ui_dst_i = _pad_edges(ui[0], ui[1])
    bi_src_i, bi_dst_b = _pad_edges(bi_flat, b_idx)
    bi_src_b, bi_dst_i = _pad_edges(b_idx, bi_flat)
    ub_src_b, ub_dst_u = _pad_edges(ub[1], ub[0])
    ub_src_u, ub_dst_b = _pad_edges(ub[0], ub[1])

    ui_u, ui_i = _run_graph(users_feature, items_feature, w_u_ui, w_i_ui,
                            ui_src_i, ui_dst_u, ui_src_u, ui_dst_i)
    bi_b, bi_i = _run_graph(bundles_feature, items_feature, w_b_bi, w_i_bi,
                            bi_src_i, bi_dst_b, bi_src_b, bi_dst_i)
    ub_u, ub_b = _run_graph(users_feature, bundles_feature, w_u_ub, w_b_ub,
                            ub_src_b, ub_dst_u, ub_src_u, ub_dst_b)

    enriched = _enrich(ui_i, deg_bi, deg_ui, deg_proj_W.T)

    nbp = -(-NB // (NCORES * NTILES * _WB)) * (NCORES * NTILES * _WB)
    bi2d = jnp.concatenate([bi_flat,
                            jnp.zeros((nbp * NT - NB * NT,), jnp.int32)
                            ]).reshape((nbp * NT) // 128, 128)
    sz_idx = jnp.clip(bundle_size, 0, NT).astype(jnp.int32)
    sz_pad = jnp.concatenate([sz_idx, jnp.zeros((nbp - NB,), jnp.int32)])
    bundle_emb = _bmean(enriched, bi2d, sz_pad, size_pe_table)[:NB]

    return (ui_u, ub_u, bi_b, bi_i, ub_b, ui_i, bundle_emb)
